# speedup vs baseline: 3.2716x; 3.2716x over previous
import jax
import jax.numpy as jnp
from jax.experimental import pallas as pl

BS = 256


def _add_body(x_ref, pos_ref, o_ref):
    o_ref[...] = x_ref[...] + pos_ref[...][None, :, :]


def kernel(x, position_embeddings):
    batch, seq_len, embed = x.shape
    pos = position_embeddings[:seq_len]
    grid = (seq_len // BS,)
    return pl.pallas_call(
        _add_body,
        grid=grid,
        in_specs=[
            pl.BlockSpec((batch, BS, embed), lambda i: (0, i, 0)),
            pl.BlockSpec((BS, embed), lambda i: (i, 0)),
        ],
        out_specs=pl.BlockSpec((batch, BS, embed), lambda i: (0, i, 0)),
        out_shape=jax.ShapeDtypeStruct((batch, seq_len, embed), x.dtype),
    )(x, pos)


# speedup vs baseline: 3.2723x; 1.0002x over previous
import jax
import jax.numpy as jnp
from jax.experimental import pallas as pl

BS = 512


def _add_body(x_ref, pos_ref, o_ref):
    o_ref[...] = x_ref[...] + pos_ref[...][None, :, :]


def kernel(x, position_embeddings):
    batch, seq_len, embed = x.shape
    pos = position_embeddings[:seq_len]
    grid = (seq_len // BS,)
    return pl.pallas_call(
        _add_body,
        grid=grid,
        in_specs=[
            pl.BlockSpec((batch, BS, embed), lambda i: (0, i, 0)),
            pl.BlockSpec((BS, embed), lambda i: (i, 0)),
        ],
        out_specs=pl.BlockSpec((batch, BS, embed), lambda i: (0, i, 0)),
        out_shape=jax.ShapeDtypeStruct((batch, seq_len, embed), x.dtype),
    )(x, pos)
